# baseline (speedup 1.0000x reference)
import functools

import jax
import jax.numpy as jnp
from jax import lax
from jax.experimental import pallas as pl
from jax.experimental.pallas import tpu as pltpu
from jax.experimental.pallas import tpu_sc as plsc

_NW = 32
_CH = 128


def _sc_gather(table, idx2d, chunks0, row_off=0, n_rows=None, nbuf=None):
    n_all = idx2d.shape[0] if n_rows is None else n_rows
    e_pad = n_all * _CH
    d = table.shape[1]
    if nbuf is None:
        nbuf = 4 if d >= 64 else 8
    t0 = chunks0 // 16
    t1 = (n_all - chunks0) // 16
    t_max = max(t0, t1)
    mesh = plsc.VectorSubcoreMesh(core_axis_name="c", subcore_axis_name="s")

    @functools.partial(
        pl.kernel, mesh=mesh,
        out_type=jax.ShapeDtypeStruct((e_pad, d), table.dtype),
        compiler_params=pltpu.CompilerParams(use_tc_tiling_on_sc=False),
        scratch_types=[
            pltpu.VMEM((t_max, _CH), jnp.int32),
        ] + [pltpu.VMEM((_CH, d), table.dtype) for _ in range(nbuf)]
          + [pltpu.SemaphoreType.DMA for _ in range(nbuf)],
    )
    def k(table_hbm, idx_hbm, out_hbm, idx_v, *bufs):
        rows = bufs[:nbuf]
        sems = bufs[nbuf:]
        cid = lax.axis_index("c")
        sid = lax.axis_index("s")

        @pl.when(cid == 0)
        def _():
            pltpu.sync_copy(idx_hbm.at[pl.ds(row_off + sid * t0, t0)],
                            idx_v.at[pl.ds(0, t0)])

        @pl.when(cid == 1)
        def _():
            pltpu.sync_copy(
                idx_hbm.at[pl.ds(row_off + chunks0 + sid * t1, t1)],
                idx_v.at[pl.ds(0, t1)])

        my_n = jnp.where(cid == 0, t0, t1)
        cbase = jnp.where(cid == 0, sid * t0, chunks0 + sid * t1)
        for b in range(nbuf):
            @pl.when(b < my_n)
            def _():
                pltpu.async_copy(table_hbm.at[idx_v.at[b]], rows[b], sems[b])

        def outer(c0, carry):
            for b in range(nbuf):
                c = c0 * nbuf + b

                @pl.when(c < my_n)
                def _():
                    pltpu.make_async_copy(
                        table_hbm.at[idx_v.at[b]], rows[b], sems[b]).wait()
                    pltpu.sync_copy(
                        rows[b], out_hbm.at[pl.ds((cbase + c) * _CH, _CH)])

                @pl.when(c + nbuf < my_n)
                def _():
                    pltpu.async_copy(
                        table_hbm.at[idx_v.at[c + nbuf]], rows[b], sems[b])
            return carry

        lax.fori_loop(0, (t_max + nbuf - 1) // nbuf, outer, 0)

    return k(table, idx2d)


def _sc_scatter_add(vals, idx2d, zeros, n_acc, row_off=0, n_rows=None):
    n_rows = idx2d.shape[0] if n_rows is None else n_rows
    e_pad = n_rows * _CH
    b_per_w = e_pad // _NW
    n_ch = b_per_w // _CH
    n_slice = n_acc // 16
    mesh = plsc.VectorSubcoreMesh(core_axis_name="c", subcore_axis_name="s")

    @functools.partial(
        pl.kernel, mesh=mesh,
        out_type=jax.ShapeDtypeStruct((2, n_acc, 16), jnp.float32),
        compiler_params=pltpu.CompilerParams(use_tc_tiling_on_sc=False),
        scratch_types=[
            pltpu.VMEM((n_ch, _CH), jnp.int32),
            pltpu.VMEM((b_per_w, 16), jnp.float32),
            pltpu.VMEM_SHARED((n_acc, 16), jnp.float32),
            pltpu.SemaphoreType.DMA,
        ],
    )
    def k(vals_hbm, idx_hbm, zeros_hbm, out_hbm, idx_v, vals_v, acc_sh, sem):
        cid = lax.axis_index("c")
        sid = lax.axis_index("s")
        wid = sid * 2 + cid
        base = wid * b_per_w
        pltpu.async_copy(vals_hbm.at[pl.ds(base, b_per_w)], vals_v, sem)
        pltpu.sync_copy(idx_hbm.at[pl.ds(row_off + wid * n_ch, n_ch)], idx_v)
        pltpu.sync_copy(zeros_hbm.at[pl.ds(sid * n_slice, n_slice)],
                        acc_sh.at[pl.ds(sid * n_slice, n_slice)])
        pltpu.make_async_copy(
            vals_hbm.at[pl.ds(base, b_per_w)], vals_v, sem).wait()
        plsc.subcore_barrier()

        def body(c, carry):
            pltpu.sync_copy(vals_v.at[pl.ds(c * _CH, _CH)],
                            acc_sh.at[idx_v.at[c]], add=True)
            return carry

        lax.fori_loop(0, n_ch, body, 0)
        plsc.subcore_barrier()
        pltpu.sync_copy(acc_sh.at[pl.ds(sid * n_slice, n_slice)],
                        out_hbm.at[cid, pl.ds(sid * n_slice, n_slice)])

    return k(vals, idx2d, zeros)


def _pack16_ref(mref, out_ref):
    for a in range(8):
        out_ref[:, 16 * a:16 * (a + 1)] = mref[a::8, :]


def _msg1_body(ea_ref, xj_ref, w1_ref, b1_ref, w2r_ref, b2r_ref, out_ref,
               mref):
    blk = xj_ref.shape[0]
    h = jnp.maximum(
        lax.dot_general(ea_ref[...], w1_ref[...], (((0,), (0,)), ((), ())),
                        preferred_element_type=jnp.float32)
        + b1_ref[...], 0.0).astype(jnp.bfloat16)
    xj = xj_ref[...]
    c = jnp.dot(h, w2r_ref[...], preferred_element_type=jnp.float32) \
        + b2r_ref[...]
    t = jnp.concatenate([xj, xj, xj, xj], axis=1) * c
    col = lax.broadcasted_iota(jnp.int32, (blk, 16), 1)
    m = jnp.where(col == 4, 1.0, 0.0)
    for o in range(4):
        rs = jnp.sum(t[:, 128 * o:128 * (o + 1)], axis=1, keepdims=True)
        m = m + rs * jnp.where(col == o, 1.0, 0.0)
    mref[...] = m
    _pack16_ref(mref, out_ref)


def _msg2_body(ea_ref, a2p_ref, w1_ref, b1_ref, w2_ref, b2_ref, out_ref,
               mref):
    h2e = jnp.maximum(
        lax.dot_general(ea_ref[...], w1_ref[...], (((0,), (0,)), ((), ())),
                        preferred_element_type=jnp.float32)
        + b1_ref[...], 0.0)
    ew2 = jnp.dot(h2e, w2_ref[...], preferred_element_type=jnp.float32) \
        + b2_ref[...]
    mref[...] = ew2
    ew2p = jnp.concatenate(
        [mref[a::8, :] for a in range(8)], axis=1)
    rr = lax.broadcasted_iota(jnp.int32, (128, 128), 0)
    cc = lax.broadcasted_iota(jnp.int32, (128, 128), 1)
    sd = jnp.where((rr // 16 == cc // 16) & (rr % 4 == cc % 16), 1.0, 0.0)
    prod = a2p_ref[...] * ew2p
    out_ref[...] = jnp.dot(prod, sd, preferred_element_type=jnp.float32)


def _unpack16_mm(pp):
    rpb = pp.shape[0]
    blk = rpb * 8
    re = lax.broadcasted_iota(jnp.int32, (blk, rpb), 0)
    ce = lax.broadcasted_iota(jnp.int32, (blk, rpb), 1)
    cj = lax.broadcasted_iota(jnp.int32, (128, 16), 0)
    jj = lax.broadcasted_iota(jnp.int32, (128, 16), 1)
    out = jnp.zeros((blk, 16), jnp.float32)
    for a in range(8):
        ea = jnp.where(re == 8 * ce + a, 1.0, 0.0)
        pa = jnp.where(cj == 16 * a + jj, 1.0, 0.0)
        out = out + jnp.dot(
            ea, jnp.dot(pp, pa, preferred_element_type=jnp.float32),
            preferred_element_type=jnp.float32)
    return out


def _combine1_body(p_ref, q_ref, x_ref, rw_ref, bias_ref, h1_ref, a2_ref,
                   sref):
    acc = _unpack16_mm(p_ref[0] + p_ref[1] + q_ref[0] + q_ref[1])
    cnt = acc[:, 4:5]
    inv = 1.0 / jnp.maximum(cnt, 1.0)
    root = jnp.dot(x_ref[...], rw_ref[...],
                   preferred_element_type=jnp.float32) + bias_ref[...]
    h1 = jnp.maximum(acc * inv + root, 0.0)
    col = lax.broadcasted_iota(jnp.int32, acc.shape, 1)
    h1 = jnp.where(col == 4, inv, jnp.where(col < 4, h1, 0.0))
    h1_ref[...] = h1
    ri = lax.broadcasted_iota(jnp.int32, (16, 16), 0)
    rj = lax.broadcasted_iota(jnp.int32, (16, 16), 1)
    r2 = jnp.where(ri == rj // 4, 1.0, 0.0)
    sref[...] = jnp.dot(h1, r2, preferred_element_type=jnp.float32)
    _pack16_ref(sref, a2_ref)


def _final_body(p_ref, q_ref, h1_ref, rw_ref, b2_ref, c1w_ref, c1b_ref,
                c2w_ref, c2b_ref, out_ref):
    acc = _unpack16_mm(p_ref[0] + p_ref[1] + q_ref[0] + q_ref[1])
    h1 = h1_ref[...]
    inv = h1[:, 4:5]
    root = jnp.dot(h1, rw_ref[...], preferred_element_type=jnp.float32)
    h2 = jnp.maximum(acc * inv + root + b2_ref[...], 0.0)
    h3 = jnp.maximum(
        jnp.dot(h2, c1w_ref[...], preferred_element_type=jnp.float32)
        + c1b_ref[...], 0.0)
    out_ref[...] = jnp.dot(h3, c2w_ref[...],
                           preferred_element_type=jnp.float32) + c2b_ref[...]


def _full_spec(shape):
    return pl.BlockSpec(shape, lambda i: tuple(0 for _ in shape))


def _msg1(ea_t, xj, w1, b1, w2r, b2r, e_pad, off=0, blk=2048):
    grid = (e_pad // blk,)
    return pl.pallas_call(
        _msg1_body,
        grid=grid,
        in_specs=[
            pl.BlockSpec((4, blk), lambda i: (0, i + off)),
            pl.BlockSpec((blk, 128), lambda i: (i, 0)),
            _full_spec((4, 512)),
            _full_spec((1, 512)),
            _full_spec((512, 512)),
            _full_spec((1, 512)),
        ],
        out_specs=pl.BlockSpec((blk // 8, 128), lambda i: (i, 0)),
        out_shape=jax.ShapeDtypeStruct((e_pad // 8, 128), jnp.float32),
        scratch_shapes=[pltpu.VMEM((blk, 16), jnp.float32)],
    )(ea_t, xj, w1, b1, w2r, b2r)


def _msg2(ea_t, a2p, w1, b1, w2, b2, e_pad, off=0, blk=4096):
    grid = (e_pad // blk,)
    return pl.pallas_call(
        _msg2_body,
        grid=grid,
        in_specs=[
            pl.BlockSpec((4, blk), lambda i: (0, i + off)),
            pl.BlockSpec((blk // 8, 128), lambda i: (i, 0)),
            _full_spec((4, 16)),
            _full_spec((1, 16)),
            _full_spec((16, 16)),
            _full_spec((1, 16)),
        ],
        out_specs=pl.BlockSpec((blk // 8, 128), lambda i: (i, 0)),
        out_shape=jax.ShapeDtypeStruct((e_pad // 8, 128), jnp.float32),
        scratch_shapes=[pltpu.VMEM((blk, 16), jnp.float32)],
    )(ea_t, a2p, w1, b1, w2, b2)


def _combine1(partial, partial_b, x_pad, rw16, b16, n_pad, blk=1024):
    grid = (n_pad // blk,)
    return pl.pallas_call(
        _combine1_body,
        grid=grid,
        in_specs=[
            pl.BlockSpec((2, blk // 8, 128), lambda i: (0, i, 0)),
            pl.BlockSpec((2, blk // 8, 128), lambda i: (0, i, 0)),
            pl.BlockSpec((blk, 128), lambda i: (i, 0)),
            _full_spec((128, 16)),
            _full_spec((1, 16)),
        ],
        out_specs=[
            pl.BlockSpec((blk, 16), lambda i: (i, 0)),
            pl.BlockSpec((blk // 8, 128), lambda i: (i, 0)),
        ],
        out_shape=[
            jax.ShapeDtypeStruct((n_pad, 16), jnp.float32),
            jax.ShapeDtypeStruct((n_pad // 8, 128), jnp.float32),
        ],
        scratch_shapes=[pltpu.VMEM((blk, 16), jnp.float32)],
    )(partial, partial_b, x_pad, rw16, b16)


def _final(partial2, partial2_b, h1p, rw16, b16, c1w, c1b, c2w, c2b, n_pad,
           blk=1024):
    grid = (n_pad // blk,)
    return pl.pallas_call(
        _final_body,
        grid=grid,
        in_specs=[
            pl.BlockSpec((2, blk // 8, 128), lambda i: (0, i, 0)),
            pl.BlockSpec((2, blk // 8, 128), lambda i: (0, i, 0)),
            pl.BlockSpec((blk, 16), lambda i: (i, 0)),
            _full_spec((16, 16)),
            _full_spec((1, 16)),
            _full_spec((16, 16)),
            _full_spec((1, 16)),
            _full_spec((16, 40)),
            _full_spec((1, 40)),
        ],
        out_specs=pl.BlockSpec((blk, 40), lambda i: (i, 0)),
        out_shape=jax.ShapeDtypeStruct((n_pad, 40), jnp.float32),
    )(partial2, partial2_b, h1p, rw16, b16, c1w, c1b, c2w, c2b)


def kernel(x, edge_index, edge_attr, nn1_w1, nn1_b1, nn1_w2, nn1_b2, root1_w,
           bias1, nn2_w1, nn2_b1, nn2_w2, nn2_b2, root2_w, bias2, cls_w1,
           cls_b1, cls_w2, cls_b2):
    n, in_ch = x.shape
    e = edge_index.shape[1]
    gran = _NW * _CH
    e_pad = ((e + gran - 1) // gran) * gran
    n_pad = ((n + 16 + 1023) // 1024) * 1024

    src = jnp.concatenate(
        [edge_index[0], jnp.zeros((e_pad - e,), jnp.int32)]
    ).reshape(e_pad // _CH, _CH)
    dst = jnp.concatenate(
        [edge_index[1], jnp.full((e_pad - e,), n, jnp.int32)]
    ).reshape(e_pad // _CH, _CH)
    ea_t = jnp.pad(edge_attr.T, ((0, 0), (0, e_pad - e)))
    zeros16 = jnp.zeros((n_pad, 16), jnp.float32)
    x_pad = jnp.concatenate([x, jnp.zeros((n_pad - n, in_ch), jnp.float32)])

    hid = root1_w.shape[1]
    w2r = jnp.concatenate(
        [nn1_w2[:, o::hid] for o in range(hid)], axis=1).astype(jnp.bfloat16)
    b2r = jnp.concatenate([nn1_b2[o::hid] for o in range(hid)])[None, :]
    rw16 = jnp.pad(root1_w, ((0, 0), (0, 16 - hid)))
    b16 = jnp.pad(bias1, (0, 16 - hid))[None, :]
    r2w16 = jnp.pad(root2_w, ((0, 16 - hid), (0, 16 - hid)))
    b2_16 = jnp.pad(bias2, (0, 16 - hid))[None, :]
    c1w16 = jnp.pad(cls_w1, ((0, 16 - hid), (0, 16 - hid)))
    c1b16 = jnp.pad(cls_b1, (0, 16 - hid))[None, :]
    c2w16 = jnp.pad(cls_w2, ((0, 16 - hid), (0, 0)))
    c2b = cls_b2[None, :]

    n_all = e_pad // _CH
    e_half = e_pad // 2
    n_half = n_all // 2
    blk1, blk2 = 2048, 4096

    parts1 = []
    for h in range(2):
        off = h * n_half
        xj_h = _sc_gather(x, src, chunks0=n_half * 4 // 5,
                          row_off=off, n_rows=n_half)
        msg_h = _msg1(ea_t, xj_h, nn1_w1, nn1_b1[None, :], w2r, b2r,
                      e_half, off=h * (e_half // blk1), blk=blk1)
        parts1.append(
            _sc_scatter_add(msg_h.reshape(e_half, 16), dst, zeros16, n_pad,
                            row_off=off, n_rows=n_half)
            .reshape(2, n_pad // 8, 128))
    h1p, a2tp = _combine1(parts1[0], parts1[1], x_pad, rw16, b16, n_pad)
    a2t = a2tp.reshape(n_pad, 16)

    parts2 = []
    for h in range(2):
        off = h * n_half
        a2_h = _sc_gather(a2t, src, chunks0=n_half * 13 // 20,
                          row_off=off, n_rows=n_half)
        msg_h = _msg2(ea_t, a2_h.reshape(e_half // 8, 128), nn2_w1,
                      nn2_b1[None, :], nn2_w2, nn2_b2[None, :], e_half,
                      off=h * (e_half // blk2), blk=blk2)
        parts2.append(
            _sc_scatter_add(msg_h.reshape(e_half, 16), dst, zeros16, n_pad,
                            row_off=off, n_rows=n_half)
            .reshape(2, n_pad // 8, 128))
    out = _final(parts2[0], parts2[1], h1p, r2w16, b2_16, c1w16, c1b16,
                 c2w16, c2b, n_pad)
    return out[:n]


# speedup vs baseline: 1.0725x; 1.0725x over previous
import functools

import jax
import jax.numpy as jnp
from jax import lax
from jax.experimental import pallas as pl
from jax.experimental.pallas import tpu as pltpu
from jax.experimental.pallas import tpu_sc as plsc

_NW = 32
_CH = 128


def _sc_gather(table, idx2d, chunks0, row_off=0, n_rows=None, nbuf=None):
    n_all = idx2d.shape[0] if n_rows is None else n_rows
    e_pad = n_all * _CH
    d = table.shape[1]
    if nbuf is None:
        nbuf = 4 if d >= 64 else 8
    t0 = chunks0 // 16
    t1 = (n_all - chunks0) // 16
    t_max = max(t0, t1)
    mesh = plsc.VectorSubcoreMesh(core_axis_name="c", subcore_axis_name="s")

    @functools.partial(
        pl.kernel, mesh=mesh,
        out_type=jax.ShapeDtypeStruct((e_pad, d), table.dtype),
        compiler_params=pltpu.CompilerParams(use_tc_tiling_on_sc=False),
        scratch_types=[
            pltpu.VMEM((t_max, _CH), jnp.int32),
        ] + [pltpu.VMEM((_CH, d), table.dtype) for _ in range(nbuf)]
          + [pltpu.SemaphoreType.DMA for _ in range(nbuf)],
    )
    def k(table_hbm, idx_hbm, out_hbm, idx_v, *bufs):
        rows = bufs[:nbuf]
        sems = bufs[nbuf:]
        cid = lax.axis_index("c")
        sid = lax.axis_index("s")

        @pl.when(cid == 0)
        def _():
            pltpu.sync_copy(idx_hbm.at[pl.ds(row_off + sid * t0, t0)],
                            idx_v.at[pl.ds(0, t0)])

        @pl.when(cid == 1)
        def _():
            pltpu.sync_copy(
                idx_hbm.at[pl.ds(row_off + chunks0 + sid * t1, t1)],
                idx_v.at[pl.ds(0, t1)])

        my_n = jnp.where(cid == 0, t0, t1)
        cbase = jnp.where(cid == 0, sid * t0, chunks0 + sid * t1)
        for b in range(nbuf):
            @pl.when(b < my_n)
            def _():
                pltpu.async_copy(table_hbm.at[idx_v.at[b]], rows[b], sems[b])

        def outer(c0, carry):
            for b in range(nbuf):
                c = c0 * nbuf + b

                @pl.when(c < my_n)
                def _():
                    pltpu.make_async_copy(
                        table_hbm.at[idx_v.at[b]], rows[b], sems[b]).wait()
                    pltpu.sync_copy(
                        rows[b], out_hbm.at[pl.ds((cbase + c) * _CH, _CH)])

                @pl.when(c + nbuf < my_n)
                def _():
                    pltpu.async_copy(
                        table_hbm.at[idx_v.at[c + nbuf]], rows[b], sems[b])
            return carry

        lax.fori_loop(0, (t_max + nbuf - 1) // nbuf, outer, 0)

    return k(table, idx2d)


def _sc_scatter_add(vals, idx2d, zeros, n_acc, row_off=0, n_rows=None):
    n_rows = idx2d.shape[0] if n_rows is None else n_rows
    e_pad = n_rows * _CH
    b_per_w = e_pad // _NW
    n_ch = b_per_w // _CH
    n_slice = n_acc // 16
    mesh = plsc.VectorSubcoreMesh(core_axis_name="c", subcore_axis_name="s")

    @functools.partial(
        pl.kernel, mesh=mesh,
        out_type=jax.ShapeDtypeStruct((2, n_acc, 16), jnp.float32),
        compiler_params=pltpu.CompilerParams(use_tc_tiling_on_sc=False),
        scratch_types=[
            pltpu.VMEM((n_ch, _CH), jnp.int32),
            pltpu.VMEM((b_per_w, 16), jnp.float32),
            pltpu.VMEM_SHARED((n_acc, 16), jnp.float32),
            pltpu.SemaphoreType.DMA,
        ],
    )
    def k(vals_hbm, idx_hbm, zeros_hbm, out_hbm, idx_v, vals_v, acc_sh, sem):
        cid = lax.axis_index("c")
        sid = lax.axis_index("s")
        wid = sid * 2 + cid
        base = wid * b_per_w
        pltpu.async_copy(vals_hbm.at[pl.ds(base, b_per_w)], vals_v, sem)
        pltpu.sync_copy(idx_hbm.at[pl.ds(row_off + wid * n_ch, n_ch)], idx_v)
        pltpu.sync_copy(zeros_hbm.at[pl.ds(sid * n_slice, n_slice)],
                        acc_sh.at[pl.ds(sid * n_slice, n_slice)])
        pltpu.make_async_copy(
            vals_hbm.at[pl.ds(base, b_per_w)], vals_v, sem).wait()
        plsc.subcore_barrier()

        def body(c, carry):
            pltpu.sync_copy(vals_v.at[pl.ds(c * _CH, _CH)],
                            acc_sh.at[idx_v.at[c]], add=True)
            return carry

        lax.fori_loop(0, n_ch, body, 0)
        plsc.subcore_barrier()
        pltpu.sync_copy(acc_sh.at[pl.ds(sid * n_slice, n_slice)],
                        out_hbm.at[cid, pl.ds(sid * n_slice, n_slice)])

    return k(vals, idx2d, zeros)


def _pack16_ref(mref, out_ref):
    for a in range(8):
        out_ref[:, 16 * a:16 * (a + 1)] = mref[a::8, :]


def _msg1_body(ea_ref, xj_ref, w1_ref, b1_ref, w2r_ref, b2r_ref, out_ref,
               mref):
    blk = xj_ref.shape[0]
    h = jnp.maximum(
        lax.dot_general(ea_ref[...], w1_ref[...], (((0,), (0,)), ((), ())),
                        preferred_element_type=jnp.float32)
        + b1_ref[...], 0.0).astype(jnp.bfloat16)
    xj = xj_ref[...]
    c = jnp.dot(h, w2r_ref[...], preferred_element_type=jnp.float32) \
        + b2r_ref[...]
    t = jnp.concatenate([xj, xj, xj, xj], axis=1) * c
    col = lax.broadcasted_iota(jnp.int32, (blk, 16), 1)
    m = jnp.where(col == 4, 1.0, 0.0)
    for o in range(4):
        rs = jnp.sum(t[:, 128 * o:128 * (o + 1)], axis=1, keepdims=True)
        m = m + rs * jnp.where(col == o, 1.0, 0.0)
    mref[...] = m
    _pack16_ref(mref, out_ref)


def _msg2_body(ea_ref, a2p_ref, w1_ref, b1_ref, w2_ref, b2_ref, out_ref,
               mref):
    h2e = jnp.maximum(
        lax.dot_general(ea_ref[...], w1_ref[...], (((0,), (0,)), ((), ())),
                        preferred_element_type=jnp.float32)
        + b1_ref[...], 0.0)
    ew2 = jnp.dot(h2e, w2_ref[...], preferred_element_type=jnp.float32) \
        + b2_ref[...]
    mref[...] = ew2
    ew2p = jnp.concatenate(
        [mref[a::8, :] for a in range(8)], axis=1)
    rr = lax.broadcasted_iota(jnp.int32, (128, 128), 0)
    cc = lax.broadcasted_iota(jnp.int32, (128, 128), 1)
    sd = jnp.where((rr // 16 == cc // 16) & (rr % 4 == cc % 16), 1.0, 0.0)
    prod = a2p_ref[...] * ew2p
    out_ref[...] = jnp.dot(prod, sd, preferred_element_type=jnp.float32)


def _unpack16_mm(pp):
    rpb = pp.shape[0]
    blk = rpb * 8
    re = lax.broadcasted_iota(jnp.int32, (blk, rpb), 0)
    ce = lax.broadcasted_iota(jnp.int32, (blk, rpb), 1)
    cj = lax.broadcasted_iota(jnp.int32, (128, 16), 0)
    jj = lax.broadcasted_iota(jnp.int32, (128, 16), 1)
    out = jnp.zeros((blk, 16), jnp.float32)
    for a in range(8):
        ea = jnp.where(re == 8 * ce + a, 1.0, 0.0)
        pa = jnp.where(cj == 16 * a + jj, 1.0, 0.0)
        out = out + jnp.dot(
            ea, jnp.dot(pp, pa, preferred_element_type=jnp.float32),
            preferred_element_type=jnp.float32)
    return out


def _combine1_body(p_ref, x_ref, rw_ref, bias_ref, h1_ref, a2_ref, sref):
    acc = _unpack16_mm(p_ref[0] + p_ref[1])
    cnt = acc[:, 4:5]
    inv = 1.0 / jnp.maximum(cnt, 1.0)
    root = jnp.dot(x_ref[...], rw_ref[...],
                   preferred_element_type=jnp.float32) + bias_ref[...]
    h1 = jnp.maximum(acc * inv + root, 0.0)
    col = lax.broadcasted_iota(jnp.int32, acc.shape, 1)
    h1 = jnp.where(col == 4, inv, jnp.where(col < 4, h1, 0.0))
    h1_ref[...] = h1
    ri = lax.broadcasted_iota(jnp.int32, (16, 16), 0)
    rj = lax.broadcasted_iota(jnp.int32, (16, 16), 1)
    r2 = jnp.where(ri == rj // 4, 1.0, 0.0)
    sref[...] = jnp.dot(h1, r2, preferred_element_type=jnp.float32)
    _pack16_ref(sref, a2_ref)


def _final_body(p_ref, h1_ref, rw_ref, b2_ref, c1w_ref, c1b_ref,
                c2w_ref, c2b_ref, out_ref):
    acc = _unpack16_mm(p_ref[0] + p_ref[1])
    h1 = h1_ref[...]
    inv = h1[:, 4:5]
    root = jnp.dot(h1, rw_ref[...], preferred_element_type=jnp.float32)
    h2 = jnp.maximum(acc * inv + root + b2_ref[...], 0.0)
    h3 = jnp.maximum(
        jnp.dot(h2, c1w_ref[...], preferred_element_type=jnp.float32)
        + c1b_ref[...], 0.0)
    out_ref[...] = jnp.dot(h3, c2w_ref[...],
                           preferred_element_type=jnp.float32) + c2b_ref[...]


def _full_spec(shape):
    return pl.BlockSpec(shape, lambda i: tuple(0 for _ in shape))


def _msg1(ea_t, xj, w1, b1, w2r, b2r, e_pad, off=0, blk=2048):
    grid = (e_pad // blk,)
    return pl.pallas_call(
        _msg1_body,
        grid=grid,
        in_specs=[
            pl.BlockSpec((4, blk), lambda i: (0, i + off)),
            pl.BlockSpec((blk, 128), lambda i: (i, 0)),
            _full_spec((4, 512)),
            _full_spec((1, 512)),
            _full_spec((512, 512)),
            _full_spec((1, 512)),
        ],
        out_specs=pl.BlockSpec((blk // 8, 128), lambda i: (i, 0)),
        out_shape=jax.ShapeDtypeStruct((e_pad // 8, 128), jnp.float32),
        scratch_shapes=[pltpu.VMEM((blk, 16), jnp.float32)],
    )(ea_t, xj, w1, b1, w2r, b2r)


def _msg2(ea_t, a2p, w1, b1, w2, b2, e_pad, off=0, blk=4096):
    grid = (e_pad // blk,)
    return pl.pallas_call(
        _msg2_body,
        grid=grid,
        in_specs=[
            pl.BlockSpec((4, blk), lambda i: (0, i + off)),
            pl.BlockSpec((blk // 8, 128), lambda i: (i, 0)),
            _full_spec((4, 16)),
            _full_spec((1, 16)),
            _full_spec((16, 16)),
            _full_spec((1, 16)),
        ],
        out_specs=pl.BlockSpec((blk // 8, 128), lambda i: (i, 0)),
        out_shape=jax.ShapeDtypeStruct((e_pad // 8, 128), jnp.float32),
        scratch_shapes=[pltpu.VMEM((blk, 16), jnp.float32)],
    )(ea_t, a2p, w1, b1, w2, b2)


def _combine1(partial, x_pad, rw16, b16, n_pad, blk=1024):
    grid = (n_pad // blk,)
    return pl.pallas_call(
        _combine1_body,
        grid=grid,
        in_specs=[
            pl.BlockSpec((2, blk // 8, 128), lambda i: (0, i, 0)),
            pl.BlockSpec((blk, 128), lambda i: (i, 0)),
            _full_spec((128, 16)),
            _full_spec((1, 16)),
        ],
        out_specs=[
            pl.BlockSpec((blk, 16), lambda i: (i, 0)),
            pl.BlockSpec((blk // 8, 128), lambda i: (i, 0)),
        ],
        out_shape=[
            jax.ShapeDtypeStruct((n_pad, 16), jnp.float32),
            jax.ShapeDtypeStruct((n_pad // 8, 128), jnp.float32),
        ],
        scratch_shapes=[pltpu.VMEM((blk, 16), jnp.float32)],
    )(partial, x_pad, rw16, b16)


def _final(partial2, h1p, rw16, b16, c1w, c1b, c2w, c2b, n_pad, blk=1024):
    grid = (n_pad // blk,)
    return pl.pallas_call(
        _final_body,
        grid=grid,
        in_specs=[
            pl.BlockSpec((2, blk // 8, 128), lambda i: (0, i, 0)),
            pl.BlockSpec((blk, 16), lambda i: (i, 0)),
            _full_spec((16, 16)),
            _full_spec((1, 16)),
            _full_spec((16, 16)),
            _full_spec((1, 16)),
            _full_spec((16, 40)),
            _full_spec((1, 40)),
        ],
        out_specs=pl.BlockSpec((blk, 40), lambda i: (i, 0)),
        out_shape=jax.ShapeDtypeStruct((n_pad, 40), jnp.float32),
    )(partial2, h1p, rw16, b16, c1w, c1b, c2w, c2b)


def kernel(x, edge_index, edge_attr, nn1_w1, nn1_b1, nn1_w2, nn1_b2, root1_w,
           bias1, nn2_w1, nn2_b1, nn2_w2, nn2_b2, root2_w, bias2, cls_w1,
           cls_b1, cls_w2, cls_b2):
    n, in_ch = x.shape
    e = edge_index.shape[1]
    gran = _NW * _CH
    e_pad = ((e + gran - 1) // gran) * gran
    n_pad = ((n + 16 + 1023) // 1024) * 1024

    src = jnp.concatenate(
        [edge_index[0], jnp.zeros((e_pad - e,), jnp.int32)]
    ).reshape(e_pad // _CH, _CH)
    dst = jnp.concatenate(
        [edge_index[1], jnp.full((e_pad - e,), n, jnp.int32)]
    ).reshape(e_pad // _CH, _CH)
    ea_t = jnp.pad(edge_attr.T, ((0, 0), (0, e_pad - e)))
    zeros16 = jnp.zeros((n_pad, 16), jnp.float32)
    x_pad = jnp.concatenate([x, jnp.zeros((n_pad - n, in_ch), jnp.float32)])

    hid = root1_w.shape[1]
    w2r = jnp.concatenate(
        [nn1_w2[:, o::hid] for o in range(hid)], axis=1).astype(jnp.bfloat16)
    b2r = jnp.concatenate([nn1_b2[o::hid] for o in range(hid)])[None, :]
    rw16 = jnp.pad(root1_w, ((0, 0), (0, 16 - hid)))
    b16 = jnp.pad(bias1, (0, 16 - hid))[None, :]
    r2w16 = jnp.pad(root2_w, ((0, 16 - hid), (0, 16 - hid)))
    b2_16 = jnp.pad(bias2, (0, 16 - hid))[None, :]
    c1w16 = jnp.pad(cls_w1, ((0, 16 - hid), (0, 16 - hid)))
    c1b16 = jnp.pad(cls_b1, (0, 16 - hid))[None, :]
    c2w16 = jnp.pad(cls_w2, ((0, 16 - hid), (0, 0)))
    c2b = cls_b2[None, :]

    n_all = e_pad // _CH
    e_half = e_pad // 2
    n_half = n_all // 2
    blk1, blk2 = 2048, 4096

    parts1 = []
    n_q = n_all // 4
    e_q = e_pad // 4
    for h in range(4):
        off = h * n_q
        xj_h = _sc_gather(x, src, chunks0=n_q * 4 // 5,
                          row_off=off, n_rows=n_q)
        msg_h = _msg1(ea_t, xj_h, nn1_w1, nn1_b1[None, :], w2r, b2r,
                      e_q, off=h * (e_q // blk1), blk=blk1)
        parts1.append(
            _sc_scatter_add(msg_h.reshape(e_q, 16), dst, zeros16, n_pad,
                            row_off=off, n_rows=n_q)
            .reshape(2, n_pad // 8, 128))
    psum1 = parts1[0] + parts1[1] + parts1[2] + parts1[3]
    h1p, a2tp = _combine1(psum1, x_pad, rw16, b16, n_pad)
    a2t = a2tp.reshape(n_pad, 16)

    parts2 = []
    for h in range(2):
        off = h * n_half
        a2_h = _sc_gather(a2t, src, chunks0=n_half * 13 // 20,
                          row_off=off, n_rows=n_half)
        msg_h = _msg2(ea_t, a2_h.reshape(e_half // 8, 128), nn2_w1,
                      nn2_b1[None, :], nn2_w2, nn2_b2[None, :], e_half,
                      off=h * (e_half // blk2), blk=blk2)
        parts2.append(
            _sc_scatter_add(msg_h.reshape(e_half, 16), dst, zeros16, n_pad,
                            row_off=off, n_rows=n_half)
            .reshape(2, n_pad // 8, 128))
    out = _final(parts2[0] + parts2[1], h1p, r2w16, b2_16, c1w16, c1b16,
                 c2w16, c2b, n_pad)
    return out[:n]
